# baseline (speedup 1.0000x reference)
import functools

import jax
import jax.numpy as jnp
from jax import lax
from jax.experimental import pallas as pl
from jax.experimental.pallas import tpu as pltpu
from jax.experimental.pallas import tpu_sc as plsc

BATCH = 32
NUM_TX = 2
NUM_PRBS = 273
NUM_SYMS = 14
RX = 8
ROW = NUM_TX * RX
HALF = NUM_PRBS * 6
CHUNK = 21
NCHUNK = NUM_PRBS // CHUNK
CROWS = CHUNK * 6
OUT_F = CHUNK * 12 * NUM_SYMS * RX


def _sc_body(h_ref, out_ref, in0, in1, stage, ob0, ob1):
    in_bufs = (in0, in1)
    out_bufs = (ob0, ob1)
    info = plsc.get_sparse_core_info()
    nc = info.num_cores
    b = lax.axis_index("s") * nc + lax.axis_index("c")

    lane = lax.iota(jnp.int32, 16)
    lane8 = jnp.bitwise_and(lane, 7)
    msel = jnp.where(lane >= 8, 16, 0)
    base_tx = (lane8, lane8 + 8)

    h_base = b * (2 * HALF * ROW)
    o_base = b * (NUM_TX * NCHUNK * OUT_F)

    def chunk_body(c, carry):
        pltpu.sync_copy(
            h_ref.at[pl.ds(h_base + (c * CROWS) * ROW, CROWS * ROW)], in0)
        pltpu.sync_copy(
            h_ref.at[pl.ds(h_base + (HALF + c * CROWS) * ROW, CROWS * ROW)],
            in1)

        def prb_body(p, carry2):
            for d in range(2):
                for q in range(3):
                    o = (p * 6 + 2 * q) * ROW
                    w = (in_bufs[d][pl.ds(o, 16)]
                         + in_bufs[d][pl.ds(o + ROW, 16)]) * 0.5
                    stage[pl.ds((q * 2 + d) * 16, 16)] = w
            for tx in range(2):
                for q in range(3):
                    u0i = base_tx[tx] + q * 32
                    u0 = plsc.load_gather(stage, [u0i])
                    m = plsc.load_gather(stage, [u0i + msel])
                    u1 = plsc.load_gather(stage, [u0i + 16])
                    vals = (u0, u0, u0, m, u1, u1, u1)
                    for s in range(4 * q, 4 * q + 4):
                        off = (p * 12 + s) * (NUM_SYMS * RX)
                        for r in range(7):
                            out_bufs[tx][pl.ds(off + r * 16, 16)] = vals[r]
            return carry2

        lax.fori_loop(0, CHUNK, prb_body, 0)
        for tx in range(2):
            pltpu.sync_copy(
                out_bufs[tx],
                out_ref.at[pl.ds(o_base + (tx * NCHUNK + c) * OUT_F, OUT_F)])
        return carry

    lax.fori_loop(0, NCHUNK, chunk_body, 0)


_sc_call = pl.kernel(
    _sc_body,
    out_type=jax.ShapeDtypeStruct((BATCH * NUM_TX * NCHUNK * OUT_F,),
                                  jnp.float32),
    mesh=plsc.VectorSubcoreMesh(core_axis_name="c", subcore_axis_name="s"),
    compiler_params=pltpu.CompilerParams(needs_layout_passes=False),
    scratch_types=[
        pltpu.VMEM((CROWS * ROW,), jnp.float32),
        pltpu.VMEM((CROWS * ROW,), jnp.float32),
        pltpu.VMEM((6 * 16,), jnp.float32),
        pltpu.VMEM((OUT_F,), jnp.float32),
        pltpu.VMEM((OUT_F,), jnp.float32),
    ],
)


def _pe_body(ofdm_ref, sc_ref, out_ref):
    sym = lax.broadcasted_iota(jnp.int32, (NUM_SYMS, 12 * NUM_PRBS), 0)
    scg = lax.broadcasted_iota(jnp.int32, (NUM_SYMS, 12 * NUM_PRBS), 1)
    sc = scg % 12
    for tx in range(2):
        td = jnp.abs(sym - ofdm_ref[tx, 0])
        td = jnp.minimum(td, jnp.abs(sym - ofdm_ref[tx, 1]))
        fd = jnp.abs(sc - sc_ref[tx, 0])
        for j in range(1, 6):
            fd = jnp.minimum(fd, jnp.abs(sc - sc_ref[tx, j]))
        for ch, x in ((0, td.astype(jnp.float32)), (1, fd.astype(jnp.float32))):
            y = x - jnp.mean(x)
            std = jnp.sqrt(jnp.mean(y * y))
            out_ref[tx, ch] = jnp.where(std > 0.0, y / std, y)


_pe_call = pl.pallas_call(
    _pe_body,
    out_shape=jax.ShapeDtypeStruct((NUM_TX, 2, NUM_SYMS, 12 * NUM_PRBS),
                                   jnp.float32),
    in_specs=[
        pl.BlockSpec(memory_space=pltpu.SMEM),
        pl.BlockSpec(memory_space=pltpu.SMEM),
    ],
)


def kernel(y, h_hat, dmrs_ofdm_pos, dmrs_subcarrier_pos):
    h_flat = h_hat.reshape(BATCH * 2 * HALF * ROW)
    out = _sc_call(h_flat)
    h_out = out.reshape(BATCH, NUM_TX, 12 * NUM_PRBS, NUM_SYMS, RX)
    pe4 = _pe_call(dmrs_ofdm_pos, dmrs_subcarrier_pos)
    pe = jnp.transpose(pe4, (0, 3, 2, 1))
    return h_out, pe
